# baseline (speedup 1.0000x reference)
import functools

import jax
import jax.numpy as jnp
from jax import lax
from jax.experimental import pallas as pl
from jax.experimental.pallas import tpu as pltpu
from jax.experimental.pallas import tpu_sc as plsc

N_ROWS = 1000000
D = 32
N_IDX = 524288

NC = 2
NS = 16
NW = NC * NS

B_PER_W = N_IDX // NW
CHUNK = 2048
NCHUNKS = B_PER_W // CHUNK

_mesh = plsc.VectorSubcoreMesh(core_axis_name="c", subcore_axis_name="s")


@functools.partial(
    pl.kernel,
    mesh=_mesh,
    out_type=jax.ShapeDtypeStruct((2 * N_IDX, D), jnp.float32),
    scratch_types=[
        pltpu.VMEM((CHUNK,), jnp.int32),
        pltpu.VMEM((CHUNK, D), jnp.float32),
        pltpu.SemaphoreType.DMA,
    ],
    compiler_params=pltpu.CompilerParams(use_tc_tiling_on_sc=False),
)
def _gather_concat(feat0_hbm, feat1_hbm, idx0_hbm, idx1_hbm, out_hbm,
                   idx_v, rows_v, sem):
    wid = lax.axis_index("s") * NC + lax.axis_index("c")
    base = wid * B_PER_W

    def do_table(tbl_hbm, idx_hbm, out_base):
        def body(j, carry):
            off = base + j * CHUNK
            pltpu.sync_copy(idx_hbm.at[pl.ds(off, CHUNK)], idx_v)
            pltpu.async_copy(tbl_hbm.at[idx_v], rows_v, sem).wait()
            pltpu.sync_copy(rows_v, out_hbm.at[pl.ds(out_base + off, CHUNK)])
            return carry
        lax.fori_loop(0, NCHUNKS, body, 0)

    do_table(feat0_hbm, idx0_hbm, 0)
    do_table(feat1_hbm, idx1_hbm, N_IDX)


def kernel(feat_0, feat_1, index_map_0, index_map_1):
    return _gather_concat(feat_0, feat_1,
                          index_map_0.astype(jnp.int32),
                          index_map_1.astype(jnp.int32))
